# baseline (speedup 1.0000x reference)
import functools

import jax
import jax.numpy as jnp
from jax.experimental import pallas as pl
from jax.experimental.pallas import tpu as pltpu

S, B, D, H, DH = 512, 32, 256, 8, 32
E = 5 * DH + 2
G = B * H
GC = G // 2
LN_EPS = 1e-5

T_BLK = 64
ROW_BLK = 512


def _ln_qkv_body(x_ref, w_ref, g_ref, b_ref, out_ref):
    x = x_ref[...]
    mu = jnp.mean(x, axis=1, keepdims=True)
    xc = x - mu
    var = jnp.mean(xc * xc, axis=1, keepdims=True)
    o = xc * jax.lax.rsqrt(var + LN_EPS) * g_ref[...] + b_ref[...]
    m = jnp.dot(o, w_ref[...], preferred_element_type=jnp.float32)
    parts = []
    for hh in range(H):
        sub = m[:, hh * E:(hh + 1) * E].reshape(ROW_BLK // B, B, E)
        parts.append(jnp.swapaxes(sub, 1, 2))
    out_ref[...] = jnp.concatenate(parts, axis=2)


def _out_body(h_ref, w_ref, x_ref, out_ref):
    blk = h_ref[...]
    parts = []
    for hh in range(H):
        sub = blk[:, :, hh * B:(hh + 1) * B]
        parts.append(jnp.swapaxes(sub, 1, 2))
    hs_tile = jnp.concatenate(parts, axis=2).reshape(ROW_BLK, H * DH)
    out_ref[...] = x_ref[...] + jnp.dot(
        hs_tile, w_ref[...], preferred_element_type=jnp.float32)


def _rep8(x2d):
    return jnp.broadcast_to(x2d.reshape(DH, 1, GC), (DH, 8, GC))


def _dot_rep(a2d, b2d):
    p = (a2d * b2d).reshape(DH // 8, 8, GC)
    s = p[0] + p[1] + p[2] + p[3]
    tot = jnp.sum(s, axis=0, keepdims=True)
    return jnp.broadcast_to(tot, (8, GC))


def _scan_body(qkv_ref, out_ref,
               W_s, R_s, qr_s, qa_s, ka_s, rka_s, bb_s, rbb_s,
               kp_s, rkp_s, dp_s, drp_s):
    sb = pl.program_id(1)

    @pl.when(sb == 0)
    def _init():
        W_s[...] = jnp.zeros_like(W_s)
        R_s[...] = jnp.zeros_like(R_s)
        qr_s[...] = jnp.full_like(qr_s, 1.0 / DH)
        kp_s[...] = jnp.zeros_like(kp_s)
        rkp_s[...] = jnp.zeros_like(rkp_s)
        dp_s[...] = jnp.zeros_like(dp_s)
        drp_s[...] = jnp.zeros_like(drp_s)

    q = qkv_ref[:, 0 * DH:1 * DH, :]
    qa = jnp.where(q > 0, q + 1.0, jnp.exp(q))
    qa_s[...] = qa / jnp.sum(qa, axis=1, keepdims=True)
    k = qkv_ref[:, 1 * DH:2 * DH, :]
    ka = jnp.where(k > 0, k + 1.0, jnp.exp(k))
    ka_s[...] = ka / jnp.sum(ka, axis=1, keepdims=True)
    rk = qkv_ref[:, 3 * DH:4 * DH, :]
    rk = rk - jnp.max(rk, axis=1, keepdims=True)
    erk = jnp.exp(rk)
    rka_s[...] = erk / jnp.sum(erk, axis=1, keepdims=True)
    bb_s[...] = jnp.broadcast_to(
        jax.nn.sigmoid(qkv_ref[:, 5 * DH, :])[:, None, :], (T_BLK, 8, GC))
    rbb_s[...] = jnp.broadcast_to(
        jax.nn.sigmoid(qkv_ref[:, 5 * DH + 1, :])[:, None, :],
        (T_BLK, 8, GC))

    def step(t, carry):
        qr, kprev, rkprev, dp, drp = carry
        kv = ka_s[t]
        qv = qa_s[t]

        accv = [jnp.zeros((DH // 8, 8, GC), jnp.float32)] * 2
        accz = [jnp.zeros((DH // 8, 8, GC), jnp.float32)] * 2
        for j in range(DH):
            kbp = jnp.broadcast_to(kprev[j:j + 1, :], (8, GC))
            kb = jnp.broadcast_to(kv[j:j + 1, :], (8, GC))
            qb = jnp.broadcast_to(qv[j:j + 1, :], (8, GC))
            w = W_s[j] + kbp[None] * dp
            W_s[j] = w
            accv[j % 2] = accv[j % 2] + w * kb[None]
            accz[j % 2] = accz[j % 2] + w * qb[None]
        v_old = accv[0] + accv[1]
        z_old = accz[0] + accz[1]

        vt = qkv_ref[t, 2 * DH:3 * DH, :].reshape(DH // 8, 8, GC)
        bt = bb_s[t][None]
        d = bt * (vt - v_old)
        kq = _dot_rep(kv, qv)
        z = z_old + d * kq[None]

        rkv = rka_s[t]
        accvr = [jnp.zeros((DH // 8, 8, GC), jnp.float32)] * 2
        acch = [jnp.zeros((DH // 8, 8, GC), jnp.float32)] * 2
        for j in range(DH):
            rkbp = jnp.broadcast_to(rkprev[j:j + 1, :], (8, GC))
            rkb = jnp.broadcast_to(rkv[j:j + 1, :], (8, GC))
            qrb = jnp.broadcast_to(qr[j:j + 1, :], (8, GC))
            r = R_s[j] + rkbp[None] * drp
            R_s[j] = r
            accvr[j % 2] = accvr[j % 2] + r * rkb[None]
            acch[j % 2] = acch[j % 2] + r * qrb[None]
        v_old_r = accvr[0] + accvr[1]
        h_old = acch[0] + acch[1]

        rvt = qkv_ref[t, 4 * DH:5 * DH, :].reshape(DH // 8, 8, GC)
        rbt = rbb_s[t][None]
        dr = rbt * (rvt - v_old_r)
        rkq = _dot_rep(rkv, qr)
        h = z + h_old + dr * rkq[None]

        out_ref[t] = h.reshape(DH, GC)

        m = jnp.max(h, axis=(0, 1), keepdims=True)
        eh = jnp.exp(h - m)
        qr_n = (eh / jnp.sum(eh, axis=(0, 1), keepdims=True))
        return qr_n.reshape(DH, GC), kv, rkv, d, dr

    qr, kprev, rkprev, dp, drp = jax.lax.fori_loop(
        0, T_BLK, step,
        (qr_s[...], kp_s[...], rkp_s[...], dp_s[...], drp_s[...]))
    qr_s[...] = qr
    kp_s[...] = kprev
    rkp_s[...] = rkprev
    dp_s[...] = dp
    drp_s[...] = drp


def kernel(x, slow_W, out_W, ln_g, ln_b):
    x2d = x.reshape(S * B, D)

    qkv_t = pl.pallas_call(
        _ln_qkv_body,
        grid=(S * B // ROW_BLK,),
        in_specs=[
            pl.BlockSpec((ROW_BLK, D), lambda i: (i, 0)),
            pl.BlockSpec((D, H * E), lambda i: (0, 0)),
            pl.BlockSpec((1, D), lambda i: (0, 0)),
            pl.BlockSpec((1, D), lambda i: (0, 0)),
        ],
        out_specs=pl.BlockSpec((ROW_BLK // B, E, G), lambda i: (i, 0, 0)),
        out_shape=jax.ShapeDtypeStruct((S, E, G), jnp.float32),
        compiler_params=pltpu.CompilerParams(
            dimension_semantics=("parallel",)),
    )(x2d, slow_W.T, ln_g.reshape(1, D), ln_b.reshape(1, D))

    f32 = jnp.float32
    hs = pl.pallas_call(
        _scan_body,
        grid=(2, S // T_BLK),
        in_specs=[pl.BlockSpec((T_BLK, E, GC), lambda c, s: (s, 0, c))],
        out_specs=pl.BlockSpec((T_BLK, DH, GC), lambda c, s: (s, 0, c)),
        out_shape=jax.ShapeDtypeStruct((S, DH, G), f32),
        scratch_shapes=[
            pltpu.VMEM((DH, DH // 8, 8, GC), f32),
            pltpu.VMEM((DH, DH // 8, 8, GC), f32),
            pltpu.VMEM((DH, GC), f32),
            pltpu.VMEM((T_BLK, DH, GC), f32),
            pltpu.VMEM((T_BLK, DH, GC), f32),
            pltpu.VMEM((T_BLK, DH, GC), f32),
            pltpu.VMEM((T_BLK, 8, GC), f32),
            pltpu.VMEM((T_BLK, 8, GC), f32),
            pltpu.VMEM((DH, GC), f32),
            pltpu.VMEM((DH, GC), f32),
            pltpu.VMEM((DH // 8, 8, GC), f32),
            pltpu.VMEM((DH // 8, 8, GC), f32),
        ],
        compiler_params=pltpu.CompilerParams(
            dimension_semantics=("parallel", "arbitrary")),
    )(qkv_t)

    y = pl.pallas_call(
        _out_body,
        grid=(S * B // ROW_BLK,),
        in_specs=[
            pl.BlockSpec((ROW_BLK // B, DH, G), lambda i: (i, 0, 0)),
            pl.BlockSpec((H * DH, D), lambda i: (0, 0)),
            pl.BlockSpec((ROW_BLK, D), lambda i: (i, 0)),
        ],
        out_specs=pl.BlockSpec((ROW_BLK, D), lambda i: (i, 0)),
        out_shape=jax.ShapeDtypeStruct((S * B, D), jnp.float32),
        compiler_params=pltpu.CompilerParams(
            dimension_semantics=("parallel",)),
    )(hs, out_W.T, x2d)

    return y.reshape(S, B, D)


# speedup vs baseline: 1.0513x; 1.0513x over previous
import functools

import jax
import jax.numpy as jnp
from jax.experimental import pallas as pl
from jax.experimental.pallas import tpu as pltpu

S, B, D, H, DH = 512, 32, 256, 8, 32
E = 5 * DH + 2
G = B * H
GC = G // 2
LN_EPS = 1e-5

T_BLK = 64
ROW_BLK = 512


def _ln_qkv_body(x_ref, w_ref, g_ref, b_ref, out_ref):
    x = x_ref[...]
    mu = jnp.mean(x, axis=1, keepdims=True)
    xc = x - mu
    var = jnp.mean(xc * xc, axis=1, keepdims=True)
    o = xc * jax.lax.rsqrt(var + LN_EPS) * g_ref[...] + b_ref[...]
    m = jnp.dot(o, w_ref[...], preferred_element_type=jnp.float32)
    parts = []
    for hh in range(H):
        sub = m[:, hh * E:(hh + 1) * E].reshape(ROW_BLK // B, B, E)
        parts.append(jnp.swapaxes(sub, 1, 2))
    out_ref[...] = jnp.concatenate(parts, axis=2)


def _out_body(h_ref, w_ref, x_ref, out_ref):
    blk = h_ref[...]
    parts = []
    for hh in range(H):
        sub = blk[:, :, hh * B:(hh + 1) * B]
        parts.append(jnp.swapaxes(sub, 1, 2))
    hs_tile = jnp.concatenate(parts, axis=2).reshape(ROW_BLK, H * DH)
    out_ref[...] = x_ref[...] + jnp.dot(
        hs_tile, w_ref[...], preferred_element_type=jnp.float32)


def _rep8(x2d):
    return jnp.broadcast_to(x2d.reshape(DH, 1, GC), (DH, 8, GC))


def _dot_rep(a2d, b2d):
    p = (a2d * b2d).reshape(DH // 8, 8, GC)
    s = p[0] + p[1] + p[2] + p[3]
    tot = jnp.sum(s, axis=0, keepdims=True)
    return jnp.broadcast_to(tot, (8, GC))


def _scan_body(qkv_ref, out_ref,
               W_s, R_s, qr_s, qa_s, ka_s, rka_s, bb_s, rbb_s,
               kp_s, rkp_s, dp_s, drp_s):
    sb = pl.program_id(1)

    @pl.when(sb == 0)
    def _init():
        W_s[...] = jnp.zeros_like(W_s)
        R_s[...] = jnp.zeros_like(R_s)
        qr_s[...] = jnp.zeros_like(qr_s)
        kp_s[...] = jnp.zeros_like(kp_s)
        rkp_s[...] = jnp.zeros_like(rkp_s)
        dp_s[...] = jnp.zeros_like(dp_s)
        drp_s[...] = jnp.zeros_like(drp_s)

    q = qkv_ref[:, 0 * DH:1 * DH, :]
    qa = jnp.where(q > 0, q + 1.0, jnp.exp(q))
    qa_s[...] = qa / jnp.sum(qa, axis=1, keepdims=True)
    k = qkv_ref[:, 1 * DH:2 * DH, :]
    ka = jnp.where(k > 0, k + 1.0, jnp.exp(k))
    ka_s[...] = ka / jnp.sum(ka, axis=1, keepdims=True)
    rk = qkv_ref[:, 3 * DH:4 * DH, :]
    rk = rk - jnp.max(rk, axis=1, keepdims=True)
    erk = jnp.exp(rk)
    rka_s[...] = erk / jnp.sum(erk, axis=1, keepdims=True)
    bb_s[...] = jnp.broadcast_to(
        jax.nn.sigmoid(qkv_ref[:, 5 * DH, :])[:, None, :], (T_BLK, 8, GC))
    rbb_s[...] = jnp.broadcast_to(
        jax.nn.sigmoid(qkv_ref[:, 5 * DH + 1, :])[:, None, :],
        (T_BLK, 8, GC))

    def step(t, carry):
        h, kprev, rkprev, dp, drp = carry
        m = jnp.max(h, axis=(0, 1), keepdims=True)
        eh = jnp.exp(h - m)
        qr = (eh / jnp.sum(eh, axis=(0, 1), keepdims=True)).reshape(DH, GC)
        kv = ka_s[t]
        qv = qa_s[t]

        accv = [jnp.zeros((DH // 8, 8, GC), jnp.float32)] * 2
        accz = [jnp.zeros((DH // 8, 8, GC), jnp.float32)] * 2
        for j in range(DH):
            kbp = jnp.broadcast_to(kprev[j:j + 1, :], (8, GC))
            kb = jnp.broadcast_to(kv[j:j + 1, :], (8, GC))
            qb = jnp.broadcast_to(qv[j:j + 1, :], (8, GC))
            w = W_s[j] + kbp[None] * dp
            W_s[j] = w
            accv[j % 2] = accv[j % 2] + w * kb[None]
            accz[j % 2] = accz[j % 2] + w * qb[None]
        v_old = accv[0] + accv[1]
        z_old = accz[0] + accz[1]

        vt = qkv_ref[t, 2 * DH:3 * DH, :].reshape(DH // 8, 8, GC)
        bt = bb_s[t][None]
        d = bt * (vt - v_old)
        kq = _dot_rep(kv, qv)
        z = z_old + d * kq[None]

        rkv = rka_s[t]
        accvr = [jnp.zeros((DH // 8, 8, GC), jnp.float32)] * 2
        acch = [jnp.zeros((DH // 8, 8, GC), jnp.float32)] * 2
        for j in range(DH):
            rkbp = jnp.broadcast_to(rkprev[j:j + 1, :], (8, GC))
            rkb = jnp.broadcast_to(rkv[j:j + 1, :], (8, GC))
            qrb = jnp.broadcast_to(qr[j:j + 1, :], (8, GC))
            r = R_s[j] + rkbp[None] * drp
            R_s[j] = r
            accvr[j % 2] = accvr[j % 2] + r * rkb[None]
            acch[j % 2] = acch[j % 2] + r * qrb[None]
        v_old_r = accvr[0] + accvr[1]
        h_old = acch[0] + acch[1]

        rvt = qkv_ref[t, 4 * DH:5 * DH, :].reshape(DH // 8, 8, GC)
        rbt = rbb_s[t][None]
        dr = rbt * (rvt - v_old_r)
        rkq = _dot_rep(rkv, qr)
        h = z + h_old + dr * rkq[None]

        out_ref[t] = h.reshape(DH, GC)
        return h, kv, rkv, d, dr

    def step2(i, carry):
        return step(2 * i + 1, step(2 * i, carry))

    h, kprev, rkprev, dp, drp = jax.lax.fori_loop(
        0, T_BLK // 2, step2,
        (qr_s[...].reshape(DH // 8, 8, GC), kp_s[...], rkp_s[...],
         dp_s[...], drp_s[...]))
    qr_s[...] = h.reshape(DH, GC)
    kp_s[...] = kprev
    rkp_s[...] = rkprev
    dp_s[...] = dp
    drp_s[...] = drp


def kernel(x, slow_W, out_W, ln_g, ln_b):
    x2d = x.reshape(S * B, D)

    qkv_t = pl.pallas_call(
        _ln_qkv_body,
        grid=(S * B // ROW_BLK,),
        in_specs=[
            pl.BlockSpec((ROW_BLK, D), lambda i: (i, 0)),
            pl.BlockSpec((D, H * E), lambda i: (0, 0)),
            pl.BlockSpec((1, D), lambda i: (0, 0)),
            pl.BlockSpec((1, D), lambda i: (0, 0)),
        ],
        out_specs=pl.BlockSpec((ROW_BLK // B, E, G), lambda i: (i, 0, 0)),
        out_shape=jax.ShapeDtypeStruct((S, E, G), jnp.float32),
        compiler_params=pltpu.CompilerParams(
            dimension_semantics=("parallel",)),
    )(x2d, slow_W.T, ln_g.reshape(1, D), ln_b.reshape(1, D))

    f32 = jnp.float32
    hs = pl.pallas_call(
        _scan_body,
        grid=(2, S // T_BLK),
        in_specs=[pl.BlockSpec((T_BLK, E, GC), lambda c, s: (s, 0, c))],
        out_specs=pl.BlockSpec((T_BLK, DH, GC), lambda c, s: (s, 0, c)),
        out_shape=jax.ShapeDtypeStruct((S, DH, G), f32),
        scratch_shapes=[
            pltpu.VMEM((DH, DH // 8, 8, GC), f32),
            pltpu.VMEM((DH, DH // 8, 8, GC), f32),
            pltpu.VMEM((DH, GC), f32),
            pltpu.VMEM((T_BLK, DH, GC), f32),
            pltpu.VMEM((T_BLK, DH, GC), f32),
            pltpu.VMEM((T_BLK, DH, GC), f32),
            pltpu.VMEM((T_BLK, 8, GC), f32),
            pltpu.VMEM((T_BLK, 8, GC), f32),
            pltpu.VMEM((DH, GC), f32),
            pltpu.VMEM((DH, GC), f32),
            pltpu.VMEM((DH // 8, 8, GC), f32),
            pltpu.VMEM((DH // 8, 8, GC), f32),
        ],
        compiler_params=pltpu.CompilerParams(
            dimension_semantics=("parallel", "arbitrary")),
    )(qkv_t)

    y = pl.pallas_call(
        _out_body,
        grid=(S * B // ROW_BLK,),
        in_specs=[
            pl.BlockSpec((ROW_BLK // B, DH, G), lambda i: (i, 0, 0)),
            pl.BlockSpec((H * DH, D), lambda i: (0, 0)),
            pl.BlockSpec((ROW_BLK, D), lambda i: (i, 0)),
        ],
        out_specs=pl.BlockSpec((ROW_BLK, D), lambda i: (i, 0)),
        out_shape=jax.ShapeDtypeStruct((S * B, D), jnp.float32),
        compiler_params=pltpu.CompilerParams(
            dimension_semantics=("parallel",)),
    )(hs, out_W.T, x2d)

    return y.reshape(S, B, D)
